# Initial kernel scaffold; baseline (speedup 1.0000x reference)
import jax
import jax.numpy as jnp
from jax.experimental import pallas as pl


def kernel(x, edge_index, edge_attr, u, batch, W1, b1, W2, b2):
    raise NotImplementedError("write your pallas kernel here")


# speedup vs baseline: 31.8949x; 31.8949x over previous
import functools

import jax
import jax.numpy as jnp
from jax import lax
from jax.experimental import pallas as pl
from jax.experimental.pallas import tpu as pltpu
from jax.experimental.pallas import tpu_sc as plsc

_NC, _NS = 2, 16
_NW = _NC * _NS
_LANES = 16


def _sc_partial_denom(col, attr, n_nodes, block_n):
    e = col.shape[0]
    chunk = e // _NW
    nb = n_nodes // block_n
    mesh = plsc.VectorSubcoreMesh(
        core_axis_name="c", subcore_axis_name="s",
        num_cores=_NC, num_subcores=_NS)

    @functools.partial(
        pl.kernel,
        out_type=jax.ShapeDtypeStruct((nb, _NW, block_n), jnp.float32),
        mesh=mesh,
        scratch_types=[
            pltpu.VMEM((chunk,), jnp.int32),
            pltpu.VMEM((chunk,), jnp.float32),
            pltpu.VMEM((n_nodes,), jnp.float32),
        ],
        compiler_params=pltpu.CompilerParams(
            needs_layout_passes=False, use_tc_tiling_on_sc=False),
    )
    def sc_kernel(col_hbm, attr_hbm, out_hbm, idx_v, val_v, acc_v):
        wid = lax.axis_index("s") * _NC + lax.axis_index("c")
        base = wid * chunk
        pltpu.sync_copy(col_hbm.at[pl.ds(base, chunk)], idx_v)
        pltpu.sync_copy(attr_hbm.at[pl.ds(base, chunk)], val_v)

        zeros = jnp.zeros((_LANES,), jnp.float32)

        def zero_body(i, _):
            acc_v[pl.ds(i * _LANES, _LANES)] = zeros
            return _

        lax.fori_loop(0, n_nodes // _LANES, zero_body, 0)

        def scat_body(i, _):
            idx = idx_v[pl.ds(i * _LANES, _LANES)]
            val = val_v[pl.ds(i * _LANES, _LANES)]
            plsc.addupdate_scatter(acc_v, [idx], val)
            return _

        lax.fori_loop(0, chunk // _LANES, scat_body, 0)
        for b in range(nb):
            pltpu.sync_copy(acc_v.at[pl.ds(b * block_n, block_n)],
                            out_hbm.at[b, wid])

    return sc_kernel(col, attr)


def _tc_masked_mlp(x, partials, w1t, b1, w2t, b2, block_n):
    n, d = x.shape
    h = w1t.shape[1]
    grid = (n // block_n,)

    def body(x_ref, part_ref, w1t_ref, b1_ref, w2t_ref, b2_ref, out_ref):
        ones = jnp.ones((_NW, 1), jnp.float32)
        denom = lax.dot_general(
            part_ref[0], ones, (((0,), (0,)), ((), ())),
            preferred_element_type=jnp.float32)
        g = jnp.where(denom != 0.0, 2.0 * x_ref[...], 0.0)
        h1 = jnp.dot(g, w1t_ref[...], preferred_element_type=jnp.float32)
        h1 = jnp.maximum(h1 + b1_ref[...], 0.0)
        r = jnp.dot(h1, w2t_ref[...], preferred_element_type=jnp.float32)
        out_ref[...] = jnp.maximum(r + b2_ref[...], 0.0)

    return pl.pallas_call(
        body,
        grid=grid,
        in_specs=[
            pl.BlockSpec((block_n, d), lambda i: (i, 0)),
            pl.BlockSpec((1, _NW, block_n), lambda i: (i, 0, 0)),
            pl.BlockSpec((d, h), lambda i: (0, 0)),
            pl.BlockSpec((1, h), lambda i: (0, 0)),
            pl.BlockSpec((h, d), lambda i: (0, 0)),
            pl.BlockSpec((1, d), lambda i: (0, 0)),
        ],
        out_specs=pl.BlockSpec((block_n, d), lambda i: (i, 0)),
        out_shape=jax.ShapeDtypeStruct((n, d), jnp.float32),
    )(x, partials, w1t, b1, w2t, b2)


def kernel(x, edge_index, edge_attr, u, batch, W1, b1, W2, b2):
    n = x.shape[0]
    col = edge_index[1]
    attr = edge_attr[:, 0]
    partials = _sc_partial_denom(col, attr, n, block_n=1000)
    return _tc_masked_mlp(
        x, partials, W1.T, b1.reshape(1, -1), W2.T, b2.reshape(1, -1),
        block_n=1000)


# speedup vs baseline: 36.8915x; 1.1567x over previous
import functools

import jax
import jax.numpy as jnp
from jax import lax
from jax.experimental import pallas as pl
from jax.experimental.pallas import tpu as pltpu
from jax.experimental.pallas import tpu_sc as plsc

_NC, _NS = 2, 16
_NW = _NC * _NS
_LANES = 16
_UNROLL = 25


def _sc_partial_denom(edge_index, edge_attr, n_nodes, block_n):
    e = edge_index.shape[1]
    chunk = e // _NW
    nb = n_nodes // block_n
    mesh = plsc.VectorSubcoreMesh(
        core_axis_name="c", subcore_axis_name="s",
        num_cores=_NC, num_subcores=_NS)

    @functools.partial(
        pl.kernel,
        out_type=jax.ShapeDtypeStruct((nb, _NW, block_n), jnp.float32),
        mesh=mesh,
        scratch_types=[
            pltpu.VMEM((chunk,), jnp.int32),
            pltpu.VMEM((chunk,), jnp.float32),
            pltpu.VMEM((n_nodes,), jnp.float32),
        ],
        compiler_params=pltpu.CompilerParams(
            needs_layout_passes=False, use_tc_tiling_on_sc=False),
    )
    def sc_kernel(ei_hbm, ea_hbm, out_hbm, idx_v, val_v, acc_v):
        wid = lax.axis_index("s") * _NC + lax.axis_index("c")
        base = wid * chunk
        pltpu.sync_copy(ei_hbm.at[1, pl.ds(base, chunk)], idx_v)
        pltpu.sync_copy(ea_hbm.at[pl.ds(base, chunk)], val_v)

        zeros = jnp.zeros((_LANES,), jnp.float32)

        def zero_body(i, _):
            for u in range(_UNROLL):
                acc_v[pl.ds((i * _UNROLL + u) * _LANES, _LANES)] = zeros
            return _

        lax.fori_loop(0, n_nodes // (_LANES * _UNROLL), zero_body, 0)

        def scat_body(i, _):
            for u in range(_UNROLL):
                off = (i * _UNROLL + u) * _LANES
                idx = idx_v[pl.ds(off, _LANES)]
                val = val_v[pl.ds(off, _LANES)]
                plsc.addupdate_scatter(acc_v, [idx], val)
            return _

        lax.fori_loop(0, chunk // (_LANES * _UNROLL), scat_body, 0)
        for b in range(nb):
            pltpu.sync_copy(acc_v.at[pl.ds(b * block_n, block_n)],
                            out_hbm.at[b, wid])

    return sc_kernel(edge_index, edge_attr.reshape(-1))


def _tc_mlp(x, w1t, b1, w2t, b2, block_n):
    n, d = x.shape
    h = w1t.shape[1]

    def body(x_ref, w1t_ref, b1_ref, w2t_ref, b2_ref, out_ref):
        g = 2.0 * x_ref[...]
        h1 = jnp.dot(g, w1t_ref[...], preferred_element_type=jnp.float32)
        h1 = jnp.maximum(h1 + b1_ref[...], 0.0)
        r = jnp.dot(h1, w2t_ref[...], preferred_element_type=jnp.float32)
        out_ref[...] = jnp.maximum(r + b2_ref[...], 0.0)

    return pl.pallas_call(
        body,
        grid=(n // block_n,),
        in_specs=[
            pl.BlockSpec((block_n, d), lambda i: (i, 0)),
            pl.BlockSpec((d, h), lambda i: (0, 0)),
            pl.BlockSpec((1, h), lambda i: (0, 0)),
            pl.BlockSpec((h, d), lambda i: (0, 0)),
            pl.BlockSpec((1, d), lambda i: (0, 0)),
        ],
        out_specs=pl.BlockSpec((block_n, d), lambda i: (i, 0)),
        out_shape=jax.ShapeDtypeStruct((n, d), jnp.float32),
    )(x, w1t, b1, w2t, b2)


def _tc_select(full, partials, b1, w2t, b2, block_n):
    n, d = full.shape
    h = w2t.shape[0]

    def body(full_ref, part_ref, b1_ref, w2t_ref, b2_ref, out_ref):
        ones = jnp.ones((_NW, 1), jnp.float32)
        denom = lax.dot_general(
            part_ref[0], ones, (((0,), (0,)), ((), ())),
            preferred_element_type=jnp.float32)
        h0 = jnp.maximum(b1_ref[...], 0.0)
        c0 = jnp.dot(h0, w2t_ref[...], preferred_element_type=jnp.float32)
        c0 = jnp.maximum(c0 + b2_ref[...], 0.0)
        out_ref[...] = jnp.where(denom != 0.0, full_ref[...], c0)

    return pl.pallas_call(
        body,
        grid=(n // block_n,),
        in_specs=[
            pl.BlockSpec((block_n, d), lambda i: (i, 0)),
            pl.BlockSpec((1, _NW, block_n), lambda i: (i, 0, 0)),
            pl.BlockSpec((1, h), lambda i: (0, 0)),
            pl.BlockSpec((h, d), lambda i: (0, 0)),
            pl.BlockSpec((1, d), lambda i: (0, 0)),
        ],
        out_specs=pl.BlockSpec((block_n, d), lambda i: (i, 0)),
        out_shape=jax.ShapeDtypeStruct((n, d), jnp.float32),
    )(full, partials, b1, w2t, b2)


def kernel(x, edge_index, edge_attr, u, batch, W1, b1, W2, b2):
    n = x.shape[0]
    block_n = 1000
    w1t, w2t = W1.T, W2.T
    b1r, b2r = b1.reshape(1, -1), b2.reshape(1, -1)
    partials = _sc_partial_denom(edge_index, edge_attr, n, block_n)
    full = _tc_mlp(x, w1t, b1r, w2t, b2r, block_n)
    return _tc_select(full, partials, b1r, w2t, b2r, block_n)


# speedup vs baseline: 41.0497x; 1.1127x over previous
import functools

import jax
import jax.numpy as jnp
from jax import lax
from jax.experimental import pallas as pl
from jax.experimental.pallas import tpu as pltpu
from jax.experimental.pallas import tpu_sc as plsc

_NC, _NS = 2, 16
_NW = _NC * _NS
_LANES = 16
_UNROLL = 25


def _sc_partial_denom(edge_index, edge_attr, n_nodes, block_n):
    e = edge_index.shape[1]
    chunk = e // _NW
    nb = n_nodes // block_n
    mesh = plsc.VectorSubcoreMesh(
        core_axis_name="c", subcore_axis_name="s",
        num_cores=_NC, num_subcores=_NS)

    @functools.partial(
        pl.kernel,
        out_type=jax.ShapeDtypeStruct((nb * _NW * block_n,), jnp.float32),
        mesh=mesh,
        scratch_types=[
            pltpu.VMEM((chunk,), jnp.int32),
            pltpu.VMEM((chunk,), jnp.float32),
            pltpu.VMEM((n_nodes,), jnp.float32),
        ],
        compiler_params=pltpu.CompilerParams(
            needs_layout_passes=False, use_tc_tiling_on_sc=False),
    )
    def sc_kernel(ei_hbm, ea_hbm, out_hbm, idx_v, val_v, acc_v):
        wid = lax.axis_index("s") * _NC + lax.axis_index("c")
        base = wid * chunk
        pltpu.sync_copy(ei_hbm.at[0, pl.ds(e + base, chunk)], idx_v)
        pltpu.sync_copy(ea_hbm.at[0, pl.ds(base, chunk)], val_v)

        zeros = jnp.zeros((_LANES,), jnp.float32)

        def zero_body(i, _):
            for u in range(_UNROLL):
                acc_v[pl.ds((i * _UNROLL + u) * _LANES, _LANES)] = zeros
            return _

        lax.fori_loop(0, n_nodes // (_LANES * _UNROLL), zero_body, 0)

        def scat_body(i, _):
            for u in range(_UNROLL):
                off = (i * _UNROLL + u) * _LANES
                idx = idx_v[pl.ds(off, _LANES)]
                val = val_v[pl.ds(off, _LANES)]
                plsc.addupdate_scatter(acc_v, [idx], val)
            return _

        lax.fori_loop(0, chunk // (_LANES * _UNROLL), scat_body, 0)
        for b in range(nb):
            off = pl.multiple_of((b * _NW + wid) * block_n, 8)
            pltpu.sync_copy(
                acc_v.at[pl.ds(b * block_n, block_n)],
                out_hbm.at[pl.ds(off, block_n)])

    flat = sc_kernel(edge_index.reshape(1, -1), edge_attr.reshape(1, -1))
    return flat.reshape(nb, _NW, block_n)


def _tc_mlp(x, w1t, b1, w2t, b2, block_n):
    n, d = x.shape
    h = w1t.shape[1]

    def body(x_ref, w1t_ref, b1_ref, w2t_ref, b2_ref, out_ref):
        g = 2.0 * x_ref[...]
        h1 = jnp.dot(g, w1t_ref[...], preferred_element_type=jnp.float32)
        h1 = jnp.maximum(h1 + b1_ref[...], 0.0)
        r = jnp.dot(h1, w2t_ref[...], preferred_element_type=jnp.float32)
        out_ref[...] = jnp.maximum(r + b2_ref[...], 0.0)

    return pl.pallas_call(
        body,
        grid=(n // block_n,),
        in_specs=[
            pl.BlockSpec((block_n, d), lambda i: (i, 0)),
            pl.BlockSpec((d, h), lambda i: (0, 0)),
            pl.BlockSpec((1, h), lambda i: (0, 0)),
            pl.BlockSpec((h, d), lambda i: (0, 0)),
            pl.BlockSpec((1, d), lambda i: (0, 0)),
        ],
        out_specs=pl.BlockSpec((block_n, d), lambda i: (i, 0)),
        out_shape=jax.ShapeDtypeStruct((n, d), jnp.float32),
    )(x, w1t, b1, w2t, b2)


def _tc_select(full, partials, b1, w2t, b2, block_n):
    n, d = full.shape
    h = w2t.shape[0]

    def body(full_ref, part_ref, b1_ref, w2t_ref, b2_ref, out_ref):
        ones = jnp.ones((_NW, 1), jnp.float32)
        denom = lax.dot_general(
            part_ref[0], ones, (((0,), (0,)), ((), ())),
            preferred_element_type=jnp.float32)
        h0 = jnp.maximum(b1_ref[...], 0.0)
        c0 = jnp.dot(h0, w2t_ref[...], preferred_element_type=jnp.float32)
        c0 = jnp.maximum(c0 + b2_ref[...], 0.0)
        out_ref[...] = jnp.where(denom != 0.0, full_ref[...], c0)

    return pl.pallas_call(
        body,
        grid=(n // block_n,),
        in_specs=[
            pl.BlockSpec((block_n, d), lambda i: (i, 0)),
            pl.BlockSpec((1, _NW, block_n), lambda i: (i, 0, 0)),
            pl.BlockSpec((1, h), lambda i: (0, 0)),
            pl.BlockSpec((h, d), lambda i: (0, 0)),
            pl.BlockSpec((1, d), lambda i: (0, 0)),
        ],
        out_specs=pl.BlockSpec((block_n, d), lambda i: (i, 0)),
        out_shape=jax.ShapeDtypeStruct((n, d), jnp.float32),
    )(full, partials, b1, w2t, b2)


def kernel(x, edge_index, edge_attr, u, batch, W1, b1, W2, b2):
    n = x.shape[0]
    block_n = 2000
    w1t, w2t = W1.T, W2.T
    b1r, b2r = b1.reshape(1, -1), b2.reshape(1, -1)
    partials = _sc_partial_denom(edge_index, edge_attr, n, block_n)
    full = _tc_mlp(x, w1t, b1r, w2t, b2r, block_n)
    return _tc_select(full, partials, b1r, w2t, b2r, block_n)
